# baseline (speedup 1.0000x reference)
import jax
import jax.numpy as jnp
from jax.experimental import pallas as pl

IMG_SIZE = 512.0


def _decode_kernel(x_ref, mul_ref, o_ref):
    x = x_ref[0]
    G = x.shape[0]
    stride = IMG_SIZE / G
    e = jnp.exp(x)
    sig = jax.nn.sigmoid(x)
    lane = jax.lax.broadcasted_iota(jnp.int32, x.shape, 2)
    f = lane % 5
    gx = jax.lax.broadcasted_iota(jnp.int32, x.shape, 1).astype(jnp.float32)
    gy = jax.lax.broadcasted_iota(jnp.int32, x.shape, 0).astype(jnp.float32)
    add = jnp.where(f == 1, gx, jnp.where(f == 2, gy, 0.0))
    smul = jnp.where(f == 0, 1.0, stride)
    res = jnp.where(f < 3, (sig + add) * smul, e * mul_ref[0])
    for a in range(x.shape[2] // 5):
        o_ref[0, a] = res[:, :, 5 * a:5 * (a + 1)]


@jax.jit
def kernel(y_pred, anchors):
    B, G, _, C = y_pred.shape
    A = anchors.shape[0]
    mul = jnp.concatenate(
        [jnp.ones((A, 3), anchors.dtype), anchors], axis=1).reshape(1, 1, C)
    return pl.pallas_call(
        _decode_kernel,
        grid=(B,),
        in_specs=[
            pl.BlockSpec((1, G, G, C), lambda b: (b, 0, 0, 0)),
            pl.BlockSpec((1, 1, C), lambda b: (0, 0, 0)),
        ],
        out_specs=pl.BlockSpec((1, A, G, G, 5), lambda b: (b, 0, 0, 0, 0)),
        out_shape=jax.ShapeDtypeStruct((B, A, G, G, 5), y_pred.dtype),
    )(y_pred, mul)


# speedup vs baseline: 2.8292x; 2.8292x over previous
import functools

import jax
import jax.numpy as jnp
import numpy as np
from jax.experimental import pallas as pl

IMG_SIZE = 512.0
LANES = 128


def _decode_kernel(x_ref, msig_ref, m1_ref, a2_ref, o_ref, *, G, C):
    x = x_ref[0]
    e = jnp.exp(x)
    d = 1.0 + e * msig_ref[...]
    res = (e * m1_ref[...]) / d + a2_ref[...]

    A = C // 5
    GC = G * C
    n_src_cols = -(-GC // LANES)
    cols = [
        jax.lax.slice(res, (0, LANES * k),
                      (G, min(LANES * (k + 1), GC)))
        for k in range(n_src_cols)
    ]
    OW = 5 * G
    u = jax.lax.broadcasted_iota(jnp.int32, (1, LANES), 1)
    for a in range(A):
        jj = np.arange(OW)
        src_np = (jj // 5) * C + (jj % 5) + 5 * a
        pieces = []
        for m in range(-(-OW // LANES)):
            lo, hi = LANES * m, min(LANES * (m + 1), OW)
            w = hi - lo
            jm = u[:, :w] + lo
            srcm = (jm // 5) * C + jm % 5 + 5 * a
            ks = np.unique(src_np[lo:hi] // LANES)
            acc = None
            for k in ks:
                idx = srcm - LANES * k
                idxb = jnp.broadcast_to(idx, (G, w))
                g = jnp.take_along_axis(cols[k], idxb, axis=1)
                if acc is None:
                    acc = g
                else:
                    msk = jnp.broadcast_to(idx >= 0, (G, w))
                    acc = jnp.where(msk, g, acc)
            pieces.append(acc)
        o_ref[0, a] = jnp.concatenate(pieces, axis=1)


@jax.jit
def kernel(y_pred, anchors):
    B, G, _, C = y_pred.shape
    A = anchors.shape[0]
    stride = IMG_SIZE / G
    x2 = y_pred.reshape(B, G, G * C)

    j = jnp.arange(G * C)
    c = j % C
    f = c % 5
    g2 = (j // C).astype(jnp.float32)
    a_idx = c // 5
    msig = jnp.where(f < 3, 1.0, 0.0).reshape(1, G * C)
    smul = jnp.where(f == 0, 1.0, stride)
    m_exp = anchors[a_idx, jnp.clip(f - 3, 0, 1)]
    m1 = jnp.where(f < 3, smul, m_exp).reshape(1, G * C)
    rows = jnp.arange(G, dtype=jnp.float32)[:, None]
    a2 = (jnp.where(f == 1, g2 * stride, 0.0)[None, :]
          + jnp.where(f == 2, stride, 0.0)[None, :] * rows)

    out = pl.pallas_call(
        functools.partial(_decode_kernel, G=G, C=C),
        grid=(B,),
        in_specs=[
            pl.BlockSpec((1, G, G * C), lambda b: (b, 0, 0)),
            pl.BlockSpec((1, G * C), lambda b: (0, 0)),
            pl.BlockSpec((1, G * C), lambda b: (0, 0)),
            pl.BlockSpec((G, G * C), lambda b: (0, 0)),
        ],
        out_specs=pl.BlockSpec((1, A, G, 5 * G), lambda b: (b, 0, 0, 0)),
        out_shape=jax.ShapeDtypeStruct((B, A, G, 5 * G), y_pred.dtype),
    )(x2, msig, m1, a2)
    return out.reshape(B, A, G, G, 5)


# speedup vs baseline: 9.4653x; 3.3456x over previous
import functools

import jax
import jax.numpy as jnp
from jax.experimental import pallas as pl

IMG_SIZE = 512.0


def _decode_kernel(x_ref, mul_ref, o_ref, *, G, C):
    stride = IMG_SIZE / G
    gx = jax.lax.broadcasted_iota(jnp.int32, (G, G), 1).astype(jnp.float32)
    gy = jax.lax.broadcasted_iota(jnp.int32, (G, G), 0).astype(jnp.float32)
    for c in range(C):
        a, f = c // 5, c % 5
        v = x_ref[0, c]
        if f < 3:
            s = jax.nn.sigmoid(v)
            if f == 0:
                r = s
            elif f == 1:
                r = (s + gx) * stride
            else:
                r = (s + gy) * stride
        else:
            r = jnp.exp(v) * mul_ref[c]
        o_ref[0, a, f] = r


@jax.jit
def kernel(y_pred, anchors):
    B, G, _, C = y_pred.shape
    A = anchors.shape[0]
    x_t = jnp.transpose(y_pred, (0, 3, 1, 2))
    mul = jnp.concatenate(
        [jnp.ones((A, 3), anchors.dtype), anchors], axis=1).reshape(C, 1, 1)
    out = pl.pallas_call(
        functools.partial(_decode_kernel, G=G, C=C),
        grid=(B,),
        in_specs=[
            pl.BlockSpec((1, C, G, G), lambda b: (b, 0, 0, 0)),
            pl.BlockSpec((C, 1, 1), lambda b: (0, 0, 0)),
        ],
        out_specs=pl.BlockSpec((1, A, 5, G, G), lambda b: (b, 0, 0, 0, 0)),
        out_shape=jax.ShapeDtypeStruct((B, A, 5, G, G), y_pred.dtype),
    )(x_t, mul)
    return jnp.transpose(out, (0, 1, 3, 4, 2))


# speedup vs baseline: 17.6357x; 1.8632x over previous
import functools

import jax
import jax.numpy as jnp
from jax.experimental import pallas as pl

IMG_SIZE = 512.0


def _decode_kernel(x_ref, mul_ref, o_ref, *, G, C, BB):
    stride = IMG_SIZE / G
    gx = jax.lax.broadcasted_iota(jnp.int32, (G, G), 1).astype(jnp.float32)
    gy = jax.lax.broadcasted_iota(jnp.int32, (G, G), 0).astype(jnp.float32)
    for bb in range(BB):
        for c in range(C):
            a, f = c // 5, c % 5
            v = x_ref[bb, c]
            if f < 3:
                s = jax.nn.sigmoid(v)
                if f == 0:
                    r = s
                elif f == 1:
                    r = (s + gx) * stride
                else:
                    r = (s + gy) * stride
            else:
                r = jnp.exp(v) * mul_ref[c]
            o_ref[bb, a, f] = r


@jax.jit
def kernel(y_pred, anchors):
    B, G, _, C = y_pred.shape
    A = anchors.shape[0]
    x_t = jnp.transpose(y_pred, (0, 3, 1, 2))
    mul = jnp.concatenate(
        [jnp.ones((A, 3), anchors.dtype), anchors], axis=1).reshape(C, 1, 1)
    BB = 4
    out = pl.pallas_call(
        functools.partial(_decode_kernel, G=G, C=C, BB=BB),
        grid=(B // BB,),
        in_specs=[
            pl.BlockSpec((BB, C, G, G), lambda b: (b, 0, 0, 0)),
            pl.BlockSpec((C, 1, 1), lambda b: (0, 0, 0)),
        ],
        out_specs=pl.BlockSpec((BB, A, 5, G, G), lambda b: (b, 0, 0, 0, 0)),
        out_shape=jax.ShapeDtypeStruct((B, A, 5, G, G), y_pred.dtype),
    )(x_t, mul)
    return jnp.transpose(out, (0, 1, 3, 4, 2))


# speedup vs baseline: 19.5273x; 1.1073x over previous
import functools

import jax
import jax.numpy as jnp
from jax.experimental import pallas as pl

IMG_SIZE = 512.0


def _decode_kernel(x_ref, mul_ref, o_ref, *, G, C, BB):
    stride = IMG_SIZE / G
    gx = jax.lax.broadcasted_iota(jnp.int32, (G, G), 1).astype(jnp.float32)
    gy = jax.lax.broadcasted_iota(jnp.int32, (G, G), 0).astype(jnp.float32)
    for bb in range(BB):
        for c in range(C):
            a, f = c // 5, c % 5
            v = x_ref[bb, c]
            if f < 3:
                s = jax.nn.sigmoid(v)
                if f == 0:
                    r = s
                elif f == 1:
                    r = (s + gx) * stride
                else:
                    r = (s + gy) * stride
            else:
                r = jnp.exp(v) * mul_ref[c]
            o_ref[bb, a, f] = r


@jax.jit
def kernel(y_pred, anchors):
    B, G, _, C = y_pred.shape
    A = anchors.shape[0]
    x_t = jnp.transpose(y_pred, (0, 3, 1, 2))
    mul = jnp.concatenate(
        [jnp.ones((A, 3), anchors.dtype), anchors], axis=1).reshape(C, 1, 1)
    BB = 8
    out = pl.pallas_call(
        functools.partial(_decode_kernel, G=G, C=C, BB=BB),
        grid=(B // BB,),
        in_specs=[
            pl.BlockSpec((BB, C, G, G), lambda b: (b, 0, 0, 0)),
            pl.BlockSpec((C, 1, 1), lambda b: (0, 0, 0)),
        ],
        out_specs=pl.BlockSpec((BB, A, 5, G, G), lambda b: (b, 0, 0, 0, 0)),
        out_shape=jax.ShapeDtypeStruct((B, A, 5, G, G), y_pred.dtype),
    )(x_t, mul)
    return jnp.transpose(out, (0, 1, 3, 4, 2))


# speedup vs baseline: 20.0215x; 1.0253x over previous
import functools

import jax
import jax.numpy as jnp
from jax.experimental import pallas as pl

IMG_SIZE = 512.0


def _decode_kernel(x_ref, mul_ref, o_ref, *, G, C, BB):
    stride = IMG_SIZE / G
    gx = jax.lax.broadcasted_iota(jnp.int32, (G, G), 1).astype(jnp.float32)
    gy = jax.lax.broadcasted_iota(jnp.int32, (G, G), 0).astype(jnp.float32)
    for bb in range(BB):
        for c in range(C):
            a, f = c // 5, c % 5
            v = x_ref[bb, c]
            if f < 3:
                s = jax.nn.sigmoid(v)
                if f == 0:
                    r = s
                elif f == 1:
                    r = (s + gx) * stride
                else:
                    r = (s + gy) * stride
            else:
                r = jnp.exp(v) * mul_ref[c]
            o_ref[bb, a, f] = r


@jax.jit
def kernel(y_pred, anchors):
    B, G, _, C = y_pred.shape
    A = anchors.shape[0]
    x_t = jnp.transpose(y_pred, (0, 3, 1, 2))
    mul = jnp.concatenate(
        [jnp.ones((A, 3), anchors.dtype), anchors], axis=1).reshape(C, 1, 1)
    BB = 16
    out = pl.pallas_call(
        functools.partial(_decode_kernel, G=G, C=C, BB=BB),
        grid=(B // BB,),
        in_specs=[
            pl.BlockSpec((BB, C, G, G), lambda b: (b, 0, 0, 0)),
            pl.BlockSpec((C, 1, 1), lambda b: (0, 0, 0)),
        ],
        out_specs=pl.BlockSpec((BB, A, 5, G, G), lambda b: (b, 0, 0, 0, 0)),
        out_shape=jax.ShapeDtypeStruct((B, A, 5, G, G), y_pred.dtype),
    )(x_t, mul)
    return jnp.transpose(out, (0, 1, 3, 4, 2))
